# baseline (speedup 1.0000x reference)
import functools

import jax
import jax.numpy as jnp
from jax import lax
from jax.experimental import pallas as pl
from jax.experimental.pallas import tpu as pltpu
from jax.experimental.pallas import tpu_sc as plsc

N = 50000
E = 800000
NPAD = 51200
EPAD = 802816
NTILES = 16
ET = EPAD // NTILES
CHUNK = 128
NCH = ET // CHUNK
NCHP = NCH // 2
RT = NPAD // NTILES
RC = 128
NRC = RT // RC
BR = 1000
GRID = N // BR
HW = 16

F32 = jnp.float32
I32 = jnp.int32


def _leaky(x, s):
    return jnp.where(x >= 0, x, s * x)


_GDN = lax.GatherDimensionNumbers(
    offset_dims=(), collapsed_slice_dims=(0,), start_index_map=(0,))


def _splat(row, k):
    idx = jnp.full((16, 1), k, I32)
    return lax.gather(row, idx, _GDN, slice_sizes=(1,),
                      mode=lax.GatherScatterMode.PROMISE_IN_BOUNDS)


def _tc1_body(des_r, tw_r, np_r, cp_r, Wd_r, bd_r, Wt_r, bt_r, Wn_r, bn_r,
              Wc_r, bc_r, Win_r, bin_r, W1_r, As_r, Ad_r,
              xw_r, als_r, ald_r, lee_r):
    d = _leaky(jnp.dot(des_r[...], Wd_r[...], preferred_element_type=F32) + bd_r[...], 0.01)
    t = _leaky(jnp.dot(tw_r[...], Wt_r[...], preferred_element_type=F32) + bt_r[...], 0.01)
    p = _leaky(jnp.dot(np_r[...], Wn_r[...], preferred_element_type=F32) + bn_r[...], 0.01)
    c = _leaky(jnp.dot(cp_r[...], Wc_r[...], preferred_element_type=F32) + bc_r[...], 0.01)
    x = jnp.concatenate([d, t, p, c], axis=1)
    x = _leaky(jnp.dot(x, Win_r[...], preferred_element_type=F32) + bin_r[...], 0.01)
    xw = jnp.dot(x, W1_r[...], preferred_element_type=F32)
    xw_r[0, :, :] = xw[:, :32]
    xw_r[1, :, :] = xw[:, 32:]
    als = jnp.dot(xw, As_r[...], preferred_element_type=F32)
    ald = jnp.dot(xw, Ad_r[...], preferred_element_type=F32)
    als_r[...] = als
    ald_r[...] = ald
    lee_r[...] = jnp.exp(_leaky(als + ald, 0.2))


def _tc1(des, tw, npr, cp, Wd, bd, Wt, bt, Wn, bn, Wc, bc, Win, bin_, W1, As, Ad):
    row = lambda i: (i, 0)
    full = lambda i: (0, 0)
    return pl.pallas_call(
        _tc1_body,
        grid=(GRID,),
        in_specs=[
            pl.BlockSpec((BR, 768), row), pl.BlockSpec((BR, 768), row),
            pl.BlockSpec((BR, 8), row), pl.BlockSpec((BR, 8), row),
            pl.BlockSpec((768, 16), full), pl.BlockSpec((1, 16), full),
            pl.BlockSpec((768, 16), full), pl.BlockSpec((1, 16), full),
            pl.BlockSpec((8, 16), full), pl.BlockSpec((1, 16), full),
            pl.BlockSpec((8, 16), full), pl.BlockSpec((1, 16), full),
            pl.BlockSpec((64, 64), full), pl.BlockSpec((1, 64), full),
            pl.BlockSpec((64, 64), full),
            pl.BlockSpec((64, HW), full), pl.BlockSpec((64, HW), full),
        ],
        out_specs=[
            pl.BlockSpec((2, BR, 32), lambda i: (0, i, 0)),
            pl.BlockSpec((BR, HW), row), pl.BlockSpec((BR, HW), row),
            pl.BlockSpec((BR, HW), row),
        ],
        out_shape=[
            jax.ShapeDtypeStruct((2, NPAD, 32), F32),
            jax.ShapeDtypeStruct((NPAD, HW), F32),
            jax.ShapeDtypeStruct((NPAD, HW), F32),
            jax.ShapeDtypeStruct((NPAD, HW), F32),
        ],
    )(des, tw, npr, cp, Wd, bd, Wt, bt, Wn, bn, Wc, bc, Win, bin_, W1, As, Ad)


def _tc2_body(x2_r, W2_r, As_r, Ad_r, xw_r, als_r, ald_r, lee_r):
    x2 = jnp.concatenate([x2_r[0, :, :], x2_r[1, :, :]], axis=1)
    xw = jnp.dot(x2, W2_r[...], preferred_element_type=F32)
    xw_r[0, :, :] = xw[:, :32]
    xw_r[1, :, :] = xw[:, 32:]
    als = jnp.dot(xw, As_r[...], preferred_element_type=F32)
    ald = jnp.dot(xw, Ad_r[...], preferred_element_type=F32)
    als_r[...] = als
    ald_r[...] = ald
    lee_r[...] = jnp.exp(_leaky(als + ald, 0.2))


def _tc2(x2_st, W2, As, Ad):
    row = lambda i: (i, 0)
    full = lambda i: (0, 0)
    return pl.pallas_call(
        _tc2_body,
        grid=(GRID,),
        in_specs=[
            pl.BlockSpec((2, BR, 32), lambda i: (0, i, 0)),
            pl.BlockSpec((64, 64), full),
            pl.BlockSpec((64, HW), full), pl.BlockSpec((64, HW), full),
        ],
        out_specs=[
            pl.BlockSpec((2, BR, 32), lambda i: (0, i, 0)),
            pl.BlockSpec((BR, HW), row), pl.BlockSpec((BR, HW), row),
            pl.BlockSpec((BR, HW), row),
        ],
        out_shape=[
            jax.ShapeDtypeStruct((2, NPAD, 32), F32),
            jax.ShapeDtypeStruct((NPAD, HW), F32),
            jax.ShapeDtypeStruct((NPAD, HW), F32),
            jax.ShapeDtypeStruct((NPAD, HW), F32),
        ],
    )(x2_st, W2, As, Ad)


_MESH = plsc.VectorSubcoreMesh(core_axis_name="c", subcore_axis_name="s")


@functools.partial(
    pl.kernel,
    out_type=[
        jax.ShapeDtypeStruct((NPAD, HW), F32),
        jax.ShapeDtypeStruct((EPAD, HW), F32),
    ],
    mesh=_MESH,
    compiler_params=pltpu.CompilerParams(use_tc_tiling_on_sc=False, skip_device_barrier=True),
    scratch_types=[
        pltpu.VMEM_SHARED((NPAD, HW), F32),
        pltpu.VMEM((4, CHUNK), I32),
        pltpu.VMEM((4, CHUNK), I32),
        pltpu.VMEM((CHUNK, HW), F32),
        pltpu.VMEM((CHUNK, HW), F32),
        pltpu.VMEM((CHUNK, HW), F32),
        pltpu.VMEM((CHUNK, HW), F32),
        pltpu.VMEM((CHUNK, HW), F32),
        pltpu.VMEM((CHUNK, HW), F32),
        pltpu.SemaphoreType.DMA((4,)),
        pltpu.SemaphoreType.DMA,
        pltpu.SemaphoreType.DMA,
        pltpu.SemaphoreType.DMA,
        pltpu.SemaphoreType.DMA,
    ],
)
def _sc_a(src_h, dst_h, als_h, ald_h, lee_h, den_h, ee_h,
          den_acc, src_i, dst_i, als_v0, als_v1, ald_v0, ald_v1,
          ee_v0, ee_v1, sem_idx, sem_g0, sem_g1, sem_o0, sem_o1):
    cid = lax.axis_index("c")
    sid = lax.axis_index("s")
    eb = sid * ET
    rb = sid * RT

    als_vs = (als_v0, als_v1)
    ald_vs = (ald_v0, ald_v1)
    ee_vs = (ee_v0, ee_v1)
    sem_gs = (sem_g0, sem_g1)
    sem_os = (sem_o0, sem_o1)

    def idx_issue(j0):
        k = jnp.bitwise_and(j0, 3)
        row = sid * NCH + j0
        pltpu.async_copy(src_h.at[pl.ds(row, 2)], src_i.at[pl.ds(k, 2)],
                         sem_idx.at[k])
        pltpu.async_copy(dst_h.at[pl.ds(row, 2)], dst_i.at[pl.ds(k, 2)],
                         sem_idx.at[k])

    def idx_wait(j0):
        k = jnp.bitwise_and(j0, 3)
        row = sid * NCH + j0
        pltpu.make_async_copy(src_h.at[pl.ds(row, 2)], src_i.at[pl.ds(k, 2)],
                              sem_idx.at[k]).wait()
        pltpu.make_async_copy(dst_h.at[pl.ds(row, 2)], dst_i.at[pl.ds(k, 2)],
                              sem_idx.at[k]).wait()

    def g_issue(j, s):
        k = jnp.bitwise_and(j, 3)
        pltpu.async_copy(als_h.at[src_i.at[k]], als_vs[s], sem_gs[s])
        pltpu.async_copy(ald_h.at[dst_i.at[k]], ald_vs[s], sem_gs[s])

    def g_wait(j, s):
        k = jnp.bitwise_and(j, 3)
        pltpu.make_async_copy(als_h.at[src_i.at[k]], als_vs[s],
                              sem_gs[s]).wait()
        pltpu.make_async_copy(ald_h.at[dst_i.at[k]], ald_vs[s],
                              sem_gs[s]).wait()

    def compute(s):
        @plsc.parallel_loop(0, CHUNK, 1, unroll=8)
        def edge_body(e):
            ev = als_vs[s][e, :] + ald_vs[s][e, :]
            ev = jnp.where(ev >= 0, ev, 0.2 * ev)
            ee_vs[s][e, :] = jnp.exp(ev)

    def edge_phase(write_den):
        def o_issue(j, s):
            if write_den:
                k = jnp.bitwise_and(j, 3)
                pltpu.async_copy(ee_vs[s], den_acc.at[dst_i.at[k]],
                                 sem_os[s], add=True)
            else:
                base = eb + j * CHUNK
                pltpu.async_copy(ee_vs[s], ee_h.at[pl.ds(base, CHUNK), :],
                                 sem_os[s])

        def o_wait(j, s):
            if write_den:
                k = jnp.bitwise_and(j, 3)
                pltpu.make_async_copy(ee_vs[s], den_acc.at[dst_i.at[k]],
                                      sem_os[s]).wait()
            else:
                base = eb + j * CHUNK
                pltpu.make_async_copy(ee_vs[s],
                                      ee_h.at[pl.ds(base, CHUNK), :],
                                      sem_os[s]).wait()

        idx_issue(jnp.int32(0))
        idx_wait(jnp.int32(0))
        g_issue(jnp.int32(0), 0)
        g_issue(jnp.int32(1), 1)

        def pair_body(jj, carry):
            j0 = jj * 2
            j1 = j0 + 1

            @pl.when(jj >= 1)
            def _():
                o_wait(j0 - 2, 0)
                o_wait(j0 - 1, 1)

            @pl.when(jj < NCHP - 1)
            def _():
                idx_issue(j0 + 2)
            g_wait(j0, 0)
            compute(0)
            o_issue(j0, 0)

            @pl.when(jj < NCHP - 1)
            def _():
                idx_wait(j0 + 2)
                g_issue(j0 + 2, 0)
            g_wait(j1, 1)
            compute(1)
            o_issue(j1, 1)

            @pl.when(jj < NCHP - 1)
            def _():
                g_issue(j0 + 3, 1)
            return carry
        lax.fori_loop(0, NCHP, pair_body, 0)
        o_wait(jnp.int32(NCH - 2), 0)
        o_wait(jnp.int32(NCH - 1), 1)

    @pl.when(cid == 0)
    def _():
        pltpu.sync_copy(lee_h.at[pl.ds(rb, RT)], den_acc.at[pl.ds(rb, RT)])
        plsc.subcore_barrier()
        edge_phase(True)
        plsc.subcore_barrier()

        def rden_body(rc, carry):
            r0 = rb + rc * RC
            pltpu.sync_copy(den_acc.at[pl.ds(r0, RC)], als_v0)

            def row_body(r, carry2):
                als_v0[r, :] = 1.0 / (als_v0[r, :] + 1e-16)
                return carry2
            lax.fori_loop(0, RC, row_body, 0)
            pltpu.sync_copy(als_v0, den_h.at[pl.ds(r0, RC)])
            return carry
        lax.fori_loop(0, NRC, rden_body, 0)

    @pl.when(cid == 1)
    def _():
        edge_phase(False)


def _make_sc_b(apply_relu):
    @functools.partial(
        pl.kernel,
        out_type=jax.ShapeDtypeStruct((2, NPAD, 32), F32),
        mesh=_MESH,
        compiler_params=pltpu.CompilerParams(use_tc_tiling_on_sc=False, skip_device_barrier=True),
        scratch_types=[
            pltpu.VMEM_SHARED((NPAD, 32), F32),
            pltpu.VMEM((4, CHUNK), I32),
            pltpu.VMEM((4, CHUNK), I32),
            pltpu.VMEM((CHUNK, 32), F32),
            pltpu.VMEM((CHUNK, 32), F32),
            pltpu.VMEM((CHUNK, 32), F32),
            pltpu.VMEM((CHUNK, 32), F32),
            pltpu.VMEM((CHUNK, HW), F32),
            pltpu.VMEM((CHUNK, HW), F32),
            pltpu.VMEM((CHUNK, HW), F32),
            pltpu.VMEM((CHUNK, HW), F32),
            pltpu.VMEM((32,), F32),
            pltpu.SemaphoreType.DMA((4,)),
            pltpu.SemaphoreType.DMA,
            pltpu.SemaphoreType.DMA,
            pltpu.SemaphoreType.DMA,
            pltpu.SemaphoreType.DMA,
        ],
    )
    def _sc_b(src_h, dst_h, xw_st_h, ee_h, den_h, lee_h, b_h, out_st_h,
              acc, src_i, dst_i, rows_v0, rows_v1, msg_v0, msg_v1,
              ee_v0, ee_v1, den_v0, den_v1, bias_v,
              sem_idx, sem_g0, sem_g1, sem_o0, sem_o1):
        cid = lax.axis_index("c")
        sid = lax.axis_index("s")

        rows_vs = (rows_v0, rows_v1)
        msg_vs = (msg_v0, msg_v1)
        ee_vs = (ee_v0, ee_v1)
        den_vs = (den_v0, den_v1)
        sem_gs = (sem_g0, sem_g1)
        sem_os = (sem_o0, sem_o1)

        def core_path(c):
            h0, h1 = 2 * c, 2 * c + 1
            xw_h = xw_st_h.at[c]
            out_h = out_st_h.at[c]
            rb = sid * RT
            eb = sid * ET

            def init_body(rc, carry):
                r0 = rb + rc * RC
                pltpu.sync_copy(xw_h.at[pl.ds(r0, RC)], rows_v0)
                pltpu.sync_copy(lee_h.at[pl.ds(r0, RC)], ee_v0)
                pltpu.sync_copy(den_h.at[pl.ds(r0, RC)], den_v0)

                def row_body(r, carry2):
                    arow = ee_v0[r, :] * den_v0[r, :]
                    a0 = _splat(arow, h0)
                    a1 = _splat(arow, h1)
                    msg_v0[r, pl.ds(0, 16)] = rows_v0[r, pl.ds(0, 16)] * a0
                    msg_v0[r, pl.ds(16, 16)] = rows_v0[r, pl.ds(16, 16)] * a1
                    return carry2
                lax.fori_loop(0, RC, row_body, 0)
                pltpu.sync_copy(msg_v0, acc.at[pl.ds(r0, RC)])
                return carry
            lax.fori_loop(0, NRC, init_body, 0)
            plsc.subcore_barrier()

            def idx_issue(j0):
                k = jnp.bitwise_and(j0, 3)
                row = sid * NCH + j0
                pltpu.async_copy(src_h.at[pl.ds(row, 2)],
                                 src_i.at[pl.ds(k, 2)], sem_idx.at[k])
                pltpu.async_copy(dst_h.at[pl.ds(row, 2)],
                                 dst_i.at[pl.ds(k, 2)], sem_idx.at[k])

            def idx_wait(j0):
                k = jnp.bitwise_and(j0, 3)
                row = sid * NCH + j0
                pltpu.make_async_copy(src_h.at[pl.ds(row, 2)],
                                      src_i.at[pl.ds(k, 2)],
                                      sem_idx.at[k]).wait()
                pltpu.make_async_copy(dst_h.at[pl.ds(row, 2)],
                                      dst_i.at[pl.ds(k, 2)],
                                      sem_idx.at[k]).wait()

            def g_issue(j, s):
                k = jnp.bitwise_and(j, 3)
                base = eb + j * CHUNK
                pltpu.async_copy(xw_h.at[src_i.at[k]], rows_vs[s], sem_gs[s])
                pltpu.async_copy(ee_h.at[pl.ds(base, CHUNK), :], ee_vs[s],
                                 sem_gs[s])
                pltpu.async_copy(den_h.at[dst_i.at[k]], den_vs[s], sem_gs[s])

            def g_wait(j, s):
                k = jnp.bitwise_and(j, 3)
                base = eb + j * CHUNK
                pltpu.make_async_copy(xw_h.at[src_i.at[k]], rows_vs[s],
                                      sem_gs[s]).wait()
                pltpu.make_async_copy(ee_h.at[pl.ds(base, CHUNK), :],
                                      ee_vs[s], sem_gs[s]).wait()
                pltpu.make_async_copy(den_h.at[dst_i.at[k]], den_vs[s],
                                      sem_gs[s]).wait()

            def o_issue(j, s):
                k = jnp.bitwise_and(j, 3)
                pltpu.async_copy(msg_vs[s], acc.at[dst_i.at[k]],
                                 sem_os[s], add=True)

            def o_wait(j, s):
                k = jnp.bitwise_and(j, 3)
                pltpu.make_async_copy(msg_vs[s], acc.at[dst_i.at[k]],
                                      sem_os[s]).wait()

            def compute(s):
                @plsc.parallel_loop(0, CHUNK, 1, unroll=16)
                def edge_body(e):
                    arow = ee_vs[s][e, :] * den_vs[s][e, :]
                    a0 = _splat(arow, h0)
                    a1 = _splat(arow, h1)
                    msg_vs[s][e, pl.ds(0, 16)] = (
                        rows_vs[s][e, pl.ds(0, 16)] * a0)
                    msg_vs[s][e, pl.ds(16, 16)] = (
                        rows_vs[s][e, pl.ds(16, 16)] * a1)

            idx_issue(jnp.int32(0))
            idx_wait(jnp.int32(0))
            g_issue(jnp.int32(0), 0)
            g_issue(jnp.int32(1), 1)

            def pair_body(jj, carry):
                j0 = jj * 2
                j1 = j0 + 1

                @pl.when(jj >= 1)
                def _():
                    o_wait(j0 - 2, 0)
                    o_wait(j0 - 1, 1)

                @pl.when(jj < NCHP - 1)
                def _():
                    idx_issue(j0 + 2)
                g_wait(j0, 0)
                compute(0)
                o_issue(j0, 0)

                @pl.when(jj < NCHP - 1)
                def _():
                    idx_wait(j0 + 2)
                    g_issue(j0 + 2, 0)
                g_wait(j1, 1)
                compute(1)
                o_issue(j1, 1)

                @pl.when(jj < NCHP - 1)
                def _():
                    g_issue(j0 + 3, 1)
                return carry
            lax.fori_loop(0, NCHP, pair_body, 0)
            o_wait(jnp.int32(NCH - 2), 0)
            o_wait(jnp.int32(NCH - 1), 1)
            plsc.subcore_barrier()

            pltpu.sync_copy(b_h.at[c], bias_v)
            bv0 = bias_v[pl.ds(0, 16)]
            bv1 = bias_v[pl.ds(16, 16)]

            def wout_body(rc, carry):
                r0 = rb + rc * RC
                pltpu.sync_copy(acc.at[pl.ds(r0, RC)], msg_v0)

                def wrow(r, carry2):
                    v0 = msg_v0[r, pl.ds(0, 16)] + bv0
                    v1 = msg_v0[r, pl.ds(16, 16)] + bv1
                    if apply_relu:
                        v0 = jnp.maximum(v0, 0.0)
                        v1 = jnp.maximum(v1, 0.0)
                    msg_v0[r, pl.ds(0, 16)] = v0
                    msg_v0[r, pl.ds(16, 16)] = v1
                    return carry2
                lax.fori_loop(0, RC, wrow, 0)
                pltpu.sync_copy(msg_v0, out_h.at[pl.ds(r0, RC)])
                return carry
            lax.fori_loop(0, NRC, wout_body, 0)

        pl.when(cid == 0)(lambda: core_path(0))
        pl.when(cid == 1)(lambda: core_path(1))

    return _sc_b


_sc_b_relu = _make_sc_b(True)
_sc_b_plain = _make_sc_b(False)


def kernel(des, tweet, num_prop, cat_prop, edge_index,
           W_des, b_des, W_tw, b_tw, W_np, b_np, W_cp, b_cp,
           W_in, b_in, W1, a_src1, a_dst1, b1, W2, a_src2, a_dst2, b2):
    np_p = jnp.pad(num_prop, ((0, 0), (0, 3)))
    cp_p = jnp.pad(cat_prop, ((0, 0), (0, 7)))
    Wn_p = jnp.pad(W_np, ((0, 3), (0, 0)))
    Wc_p = jnp.pad(W_cp, ((0, 7), (0, 0)))

    eye4 = jnp.eye(4, dtype=F32)
    As1 = jnp.pad((a_src1[:, :, None] * eye4[:, None, :]).reshape(64, 4),
                  ((0, 0), (0, HW - 4)))
    Ad1 = jnp.pad((a_dst1[:, :, None] * eye4[:, None, :]).reshape(64, 4),
                  ((0, 0), (0, HW - 4)))
    As2 = jnp.pad(jnp.tile(a_src2.reshape(64, 1), (1, 4)),
                  ((0, 0), (0, HW - 4)))
    Ad2 = jnp.pad(jnp.tile(a_dst2.reshape(64, 1), (1, 4)),
                  ((0, 0), (0, HW - 4)))

    src = jnp.pad(edge_index[0], (0, EPAD - E),
                  constant_values=N).reshape(EPAD // CHUNK, CHUNK)
    dst = jnp.pad(edge_index[1], (0, EPAD - E),
                  constant_values=N).reshape(EPAD // CHUNK, CHUNK)

    b1_st = b1.reshape(2, 32)
    b2_st = b2.reshape(2, 32)

    xw1_st, als1, ald1, lee1 = _tc1(
        des, tweet, np_p, cp_p,
        W_des, b_des.reshape(1, 16), W_tw, b_tw.reshape(1, 16),
        Wn_p, b_np.reshape(1, 16), Wc_p, b_cp.reshape(1, 16),
        W_in, b_in.reshape(1, 64), W1, As1, Ad1)

    rden1, ee1 = _sc_a(src, dst, als1, ald1, lee1)
    x2_st = _sc_b_relu(src, dst, xw1_st, ee1, rden1, lee1, b1_st)

    xw2_st, als2, ald2, lee2 = _tc2(x2_st, W2, As2, Ad2)
    rden2, ee2 = _sc_a(src, dst, als2, ald2, lee2)
    z_st = _sc_b_plain(src, dst, xw2_st, ee2, rden2, lee2, b2_st)

    return jnp.concatenate([z_st[0, :N], z_st[1, :N]], axis=1)


# speedup vs baseline: 1.0826x; 1.0826x over previous
import functools

import jax
import jax.numpy as jnp
from jax import lax
from jax.experimental import pallas as pl
from jax.experimental.pallas import tpu as pltpu
from jax.experimental.pallas import tpu_sc as plsc

N = 50000
E = 800000
NPAD = 51200
EPAD = 802816
NTILES = 16
ET = EPAD // NTILES
CHUNK = 128
NCH = ET // CHUNK
NCHP = NCH // 2
RT = NPAD // NTILES
RC = 128
NRC = RT // RC
BR = 1000
GRID = N // BR
HW = 16

F32 = jnp.float32
I32 = jnp.int32


def _leaky(x, s):
    return jnp.where(x >= 0, x, s * x)


_GDN = lax.GatherDimensionNumbers(
    offset_dims=(), collapsed_slice_dims=(0,), start_index_map=(0,))


def _splat(row, k):
    idx = jnp.full((16, 1), k, I32)
    return lax.gather(row, idx, _GDN, slice_sizes=(1,),
                      mode=lax.GatherScatterMode.PROMISE_IN_BOUNDS)


def _tc1_body(des_r, tw_r, np_r, cp_r, Wd_r, bd_r, Wt_r, bt_r, Wn_r, bn_r,
              Wc_r, bc_r, Win_r, bin_r, W1_r, As_r, Ad_r,
              xw_r, als_r, ald_r, lee_r):
    d = _leaky(jnp.dot(des_r[...], Wd_r[...], preferred_element_type=F32) + bd_r[...], 0.01)
    t = _leaky(jnp.dot(tw_r[...], Wt_r[...], preferred_element_type=F32) + bt_r[...], 0.01)
    p = _leaky(jnp.dot(np_r[...], Wn_r[...], preferred_element_type=F32) + bn_r[...], 0.01)
    c = _leaky(jnp.dot(cp_r[...], Wc_r[...], preferred_element_type=F32) + bc_r[...], 0.01)
    x = jnp.concatenate([d, t, p, c], axis=1)
    x = _leaky(jnp.dot(x, Win_r[...], preferred_element_type=F32) + bin_r[...], 0.01)
    xw = jnp.dot(x, W1_r[...], preferred_element_type=F32)
    xw_r[0, :, :] = xw[:, :32]
    xw_r[1, :, :] = xw[:, 32:]
    als = jnp.dot(xw, As_r[...], preferred_element_type=F32)
    ald = jnp.dot(xw, Ad_r[...], preferred_element_type=F32)
    als_r[...] = als
    ald_r[...] = ald
    lee_r[...] = jnp.exp(_leaky(als + ald, 0.2))


def _tc1(des, tw, npr, cp, Wd, bd, Wt, bt, Wn, bn, Wc, bc, Win, bin_, W1, As, Ad):
    row = lambda i: (i, 0)
    full = lambda i: (0, 0)
    return pl.pallas_call(
        _tc1_body,
        grid=(GRID,),
        in_specs=[
            pl.BlockSpec((BR, 768), row), pl.BlockSpec((BR, 768), row),
            pl.BlockSpec((BR, 8), row), pl.BlockSpec((BR, 8), row),
            pl.BlockSpec((768, 16), full), pl.BlockSpec((1, 16), full),
            pl.BlockSpec((768, 16), full), pl.BlockSpec((1, 16), full),
            pl.BlockSpec((8, 16), full), pl.BlockSpec((1, 16), full),
            pl.BlockSpec((8, 16), full), pl.BlockSpec((1, 16), full),
            pl.BlockSpec((64, 64), full), pl.BlockSpec((1, 64), full),
            pl.BlockSpec((64, 64), full),
            pl.BlockSpec((64, HW), full), pl.BlockSpec((64, HW), full),
        ],
        out_specs=[
            pl.BlockSpec((2, BR, 32), lambda i: (0, i, 0)),
            pl.BlockSpec((BR, HW), row), pl.BlockSpec((BR, HW), row),
            pl.BlockSpec((BR, HW), row),
        ],
        out_shape=[
            jax.ShapeDtypeStruct((2, NPAD, 32), F32),
            jax.ShapeDtypeStruct((NPAD, HW), F32),
            jax.ShapeDtypeStruct((NPAD, HW), F32),
            jax.ShapeDtypeStruct((NPAD, HW), F32),
        ],
    )(des, tw, npr, cp, Wd, bd, Wt, bt, Wn, bn, Wc, bc, Win, bin_, W1, As, Ad)


def _tc2_body(x2_r, W2_r, As_r, Ad_r, xw_r, als_r, ald_r, lee_r):
    x2 = jnp.concatenate([x2_r[0, :, :], x2_r[1, :, :]], axis=1)
    xw = jnp.dot(x2, W2_r[...], preferred_element_type=F32)
    xw_r[0, :, :] = xw[:, :32]
    xw_r[1, :, :] = xw[:, 32:]
    als = jnp.dot(xw, As_r[...], preferred_element_type=F32)
    ald = jnp.dot(xw, Ad_r[...], preferred_element_type=F32)
    als_r[...] = als
    ald_r[...] = ald
    lee_r[...] = jnp.exp(_leaky(als + ald, 0.2))


def _tc2(x2_st, W2, As, Ad):
    row = lambda i: (i, 0)
    full = lambda i: (0, 0)
    return pl.pallas_call(
        _tc2_body,
        grid=(GRID,),
        in_specs=[
            pl.BlockSpec((2, BR, 32), lambda i: (0, i, 0)),
            pl.BlockSpec((64, 64), full),
            pl.BlockSpec((64, HW), full), pl.BlockSpec((64, HW), full),
        ],
        out_specs=[
            pl.BlockSpec((2, BR, 32), lambda i: (0, i, 0)),
            pl.BlockSpec((BR, HW), row), pl.BlockSpec((BR, HW), row),
            pl.BlockSpec((BR, HW), row),
        ],
        out_shape=[
            jax.ShapeDtypeStruct((2, NPAD, 32), F32),
            jax.ShapeDtypeStruct((NPAD, HW), F32),
            jax.ShapeDtypeStruct((NPAD, HW), F32),
            jax.ShapeDtypeStruct((NPAD, HW), F32),
        ],
    )(x2_st, W2, As, Ad)


_MESH = plsc.VectorSubcoreMesh(core_axis_name="c", subcore_axis_name="s")


@functools.partial(
    pl.kernel,
    out_type=[
        jax.ShapeDtypeStruct((NPAD, HW), F32),
        jax.ShapeDtypeStruct((EPAD, HW), F32),
    ],
    mesh=_MESH,
    compiler_params=pltpu.CompilerParams(use_tc_tiling_on_sc=False),
    scratch_types=[
        pltpu.VMEM_SHARED((NPAD, HW), F32),
        pltpu.VMEM((8, CHUNK), I32),
        pltpu.VMEM((8, CHUNK), I32),
        pltpu.VMEM((CHUNK, HW), F32),
        pltpu.VMEM((CHUNK, HW), F32),
        pltpu.VMEM((CHUNK, HW), F32),
        pltpu.VMEM((CHUNK, HW), F32),
        pltpu.VMEM((CHUNK, HW), F32),
        pltpu.VMEM((CHUNK, HW), F32),
        pltpu.VMEM((CHUNK, HW), F32),
        pltpu.VMEM((CHUNK, HW), F32),
        pltpu.VMEM((CHUNK, HW), F32),
        pltpu.VMEM((CHUNK, HW), F32),
        pltpu.VMEM((CHUNK, HW), F32),
        pltpu.VMEM((CHUNK, HW), F32),
        pltpu.SemaphoreType.DMA((8,)),
        pltpu.SemaphoreType.DMA((4,)),
        pltpu.SemaphoreType.DMA((4,)),
    ],
)
def _sc_a(src_h, dst_h, als_h, ald_h, lee_h, den_h, ee_h,
          den_acc, src_i, dst_i,
          als_v0, als_v1, als_v2, als_v3,
          ald_v0, ald_v1, ald_v2, ald_v3,
          ee_v0, ee_v1, ee_v2, ee_v3,
          sem_idx, sem_g, sem_o):
    cid = lax.axis_index("c")
    sid = lax.axis_index("s")
    eb = sid * ET
    rb = sid * RT

    als_vs = (als_v0, als_v1, als_v2, als_v3)
    ald_vs = (ald_v0, ald_v1, ald_v2, ald_v3)
    ee_vs = (ee_v0, ee_v1, ee_v2, ee_v3)

    def idx_issue(j0):
        k = jnp.bitwise_and(j0, 7)
        row = sid * NCH + j0
        pltpu.async_copy(src_h.at[pl.ds(row, 2)], src_i.at[pl.ds(k, 2)],
                         sem_idx.at[k])
        pltpu.async_copy(dst_h.at[pl.ds(row, 2)], dst_i.at[pl.ds(k, 2)],
                         sem_idx.at[k])

    def idx_wait(j0):
        k = jnp.bitwise_and(j0, 7)
        row = sid * NCH + j0
        pltpu.make_async_copy(src_h.at[pl.ds(row, 2)], src_i.at[pl.ds(k, 2)],
                              sem_idx.at[k]).wait()
        pltpu.make_async_copy(dst_h.at[pl.ds(row, 2)], dst_i.at[pl.ds(k, 2)],
                              sem_idx.at[k]).wait()

    def g_issue(j, s):
        k = jnp.bitwise_and(j, 7)
        pltpu.async_copy(als_h.at[src_i.at[k]], als_vs[s], sem_g.at[s])
        pltpu.async_copy(ald_h.at[dst_i.at[k]], ald_vs[s], sem_g.at[s])

    def g_wait(j, s):
        k = jnp.bitwise_and(j, 7)
        pltpu.make_async_copy(als_h.at[src_i.at[k]], als_vs[s],
                              sem_g.at[s]).wait()
        pltpu.make_async_copy(ald_h.at[dst_i.at[k]], ald_vs[s],
                              sem_g.at[s]).wait()

    def compute(s):
        @plsc.parallel_loop(0, CHUNK, 1, unroll=8)
        def edge_body(e):
            ev = als_vs[s][e, :] + ald_vs[s][e, :]
            ev = jnp.where(ev >= 0, ev, 0.2 * ev)
            ee_vs[s][e, :] = jnp.exp(ev)

    NQ = NCH // 4

    def edge_phase(write_den):
        def o_issue(j, s):
            if write_den:
                k = jnp.bitwise_and(j, 7)
                pltpu.async_copy(ee_vs[s], den_acc.at[dst_i.at[k]],
                                 sem_o.at[s], add=True)
            else:
                base = eb + j * CHUNK
                pltpu.async_copy(ee_vs[s], ee_h.at[pl.ds(base, CHUNK), :],
                                 sem_o.at[s])

        def o_wait(j, s):
            if write_den:
                k = jnp.bitwise_and(j, 7)
                pltpu.make_async_copy(ee_vs[s], den_acc.at[dst_i.at[k]],
                                      sem_o.at[s]).wait()
            else:
                base = eb + j * CHUNK
                pltpu.make_async_copy(ee_vs[s],
                                      ee_h.at[pl.ds(base, CHUNK), :],
                                      sem_o.at[s]).wait()

        idx_issue(jnp.int32(0))
        idx_issue(jnp.int32(2))
        idx_wait(jnp.int32(0))
        g_issue(jnp.int32(0), 0)
        g_issue(jnp.int32(1), 1)
        idx_wait(jnp.int32(2))
        g_issue(jnp.int32(2), 2)
        g_issue(jnp.int32(3), 3)

        def quad_body(jj, carry):
            q0 = jj * 4

            @pl.when(jj >= 1)
            def _():
                for s in range(4):
                    o_wait(q0 - 4 + s, s)

            @pl.when(jj < NQ - 1)
            def _():
                idx_issue(q0 + 4)
                idx_issue(q0 + 6)

            for s in range(4):
                g_wait(q0 + s, s)
                compute(s)
                o_issue(q0 + s, s)
                if s == 0 or s == 2:
                    @pl.when(jj < NQ - 1)
                    def _():
                        idx_wait(q0 + 4 + s)

                @pl.when(jj < NQ - 1)
                def _():
                    g_issue(q0 + 4 + s, s)
            return carry
        lax.fori_loop(0, NQ, quad_body, 0)
        for s in range(4):
            o_wait(jnp.int32(NCH - 4 + s), s)

    @pl.when(cid == 0)
    def _():
        pltpu.sync_copy(lee_h.at[pl.ds(rb, RT)], den_acc.at[pl.ds(rb, RT)])
        plsc.subcore_barrier()
        edge_phase(True)
        plsc.subcore_barrier()

        def rden_body(rc, carry):
            r0 = rb + rc * RC
            pltpu.sync_copy(den_acc.at[pl.ds(r0, RC)], als_v0)

            def row_body(r, carry2):
                als_v0[r, :] = 1.0 / (als_v0[r, :] + 1e-16)
                return carry2
            lax.fori_loop(0, RC, row_body, 0)
            pltpu.sync_copy(als_v0, den_h.at[pl.ds(r0, RC)])
            return carry
        lax.fori_loop(0, NRC, rden_body, 0)

    @pl.when(cid == 1)
    def _():
        edge_phase(False)


def _make_sc_b(apply_relu):
    @functools.partial(
        pl.kernel,
        out_type=jax.ShapeDtypeStruct((2, NPAD, 32), F32),
        mesh=_MESH,
        compiler_params=pltpu.CompilerParams(use_tc_tiling_on_sc=False),
        scratch_types=[
            pltpu.VMEM_SHARED((NPAD, 32), F32),
            pltpu.VMEM((4, CHUNK), I32),
            pltpu.VMEM((4, CHUNK), I32),
            pltpu.VMEM((CHUNK, 32), F32),
            pltpu.VMEM((CHUNK, 32), F32),
            pltpu.VMEM((CHUNK, 32), F32),
            pltpu.VMEM((CHUNK, 32), F32),
            pltpu.VMEM((CHUNK, HW), F32),
            pltpu.VMEM((CHUNK, HW), F32),
            pltpu.VMEM((CHUNK, HW), F32),
            pltpu.VMEM((CHUNK, HW), F32),
            pltpu.VMEM((32,), F32),
            pltpu.SemaphoreType.DMA((4,)),
            pltpu.SemaphoreType.DMA,
            pltpu.SemaphoreType.DMA,
            pltpu.SemaphoreType.DMA,
            pltpu.SemaphoreType.DMA,
        ],
    )
    def _sc_b(src_h, dst_h, xw_st_h, ee_h, den_h, lee_h, b_h, out_st_h,
              acc, src_i, dst_i, rows_v0, rows_v1, msg_v0, msg_v1,
              ee_v0, ee_v1, den_v0, den_v1, bias_v,
              sem_idx, sem_g0, sem_g1, sem_o0, sem_o1):
        cid = lax.axis_index("c")
        sid = lax.axis_index("s")

        rows_vs = (rows_v0, rows_v1)
        msg_vs = (msg_v0, msg_v1)
        ee_vs = (ee_v0, ee_v1)
        den_vs = (den_v0, den_v1)
        sem_gs = (sem_g0, sem_g1)
        sem_os = (sem_o0, sem_o1)

        def core_path(c):
            h0, h1 = 2 * c, 2 * c + 1
            xw_h = xw_st_h.at[c]
            out_h = out_st_h.at[c]
            rb = sid * RT
            eb = sid * ET

            def init_body(rc, carry):
                r0 = rb + rc * RC
                pltpu.sync_copy(xw_h.at[pl.ds(r0, RC)], rows_v0)
                pltpu.sync_copy(lee_h.at[pl.ds(r0, RC)], ee_v0)
                pltpu.sync_copy(den_h.at[pl.ds(r0, RC)], den_v0)

                def row_body(r, carry2):
                    arow = ee_v0[r, :] * den_v0[r, :]
                    a0 = _splat(arow, h0)
                    a1 = _splat(arow, h1)
                    msg_v0[r, pl.ds(0, 16)] = rows_v0[r, pl.ds(0, 16)] * a0
                    msg_v0[r, pl.ds(16, 16)] = rows_v0[r, pl.ds(16, 16)] * a1
                    return carry2
                lax.fori_loop(0, RC, row_body, 0)
                pltpu.sync_copy(msg_v0, acc.at[pl.ds(r0, RC)])
                return carry
            lax.fori_loop(0, NRC, init_body, 0)
            plsc.subcore_barrier()

            def idx_issue(j0):
                k = jnp.bitwise_and(j0, 3)
                row = sid * NCH + j0
                pltpu.async_copy(src_h.at[pl.ds(row, 2)],
                                 src_i.at[pl.ds(k, 2)], sem_idx.at[k])
                pltpu.async_copy(dst_h.at[pl.ds(row, 2)],
                                 dst_i.at[pl.ds(k, 2)], sem_idx.at[k])

            def idx_wait(j0):
                k = jnp.bitwise_and(j0, 3)
                row = sid * NCH + j0
                pltpu.make_async_copy(src_h.at[pl.ds(row, 2)],
                                      src_i.at[pl.ds(k, 2)],
                                      sem_idx.at[k]).wait()
                pltpu.make_async_copy(dst_h.at[pl.ds(row, 2)],
                                      dst_i.at[pl.ds(k, 2)],
                                      sem_idx.at[k]).wait()

            def g_issue(j, s):
                k = jnp.bitwise_and(j, 3)
                base = eb + j * CHUNK
                pltpu.async_copy(xw_h.at[src_i.at[k]], rows_vs[s], sem_gs[s])
                pltpu.async_copy(ee_h.at[pl.ds(base, CHUNK), :], ee_vs[s],
                                 sem_gs[s])
                pltpu.async_copy(den_h.at[dst_i.at[k]], den_vs[s], sem_gs[s])

            def g_wait(j, s):
                k = jnp.bitwise_and(j, 3)
                base = eb + j * CHUNK
                pltpu.make_async_copy(xw_h.at[src_i.at[k]], rows_vs[s],
                                      sem_gs[s]).wait()
                pltpu.make_async_copy(ee_h.at[pl.ds(base, CHUNK), :],
                                      ee_vs[s], sem_gs[s]).wait()
                pltpu.make_async_copy(den_h.at[dst_i.at[k]], den_vs[s],
                                      sem_gs[s]).wait()

            def o_issue(j, s):
                k = jnp.bitwise_and(j, 3)
                pltpu.async_copy(msg_vs[s], acc.at[dst_i.at[k]],
                                 sem_os[s], add=True)

            def o_wait(j, s):
                k = jnp.bitwise_and(j, 3)
                pltpu.make_async_copy(msg_vs[s], acc.at[dst_i.at[k]],
                                      sem_os[s]).wait()

            def compute(s):
                @plsc.parallel_loop(0, CHUNK, 1, unroll=16)
                def edge_body(e):
                    arow = ee_vs[s][e, :] * den_vs[s][e, :]
                    a0 = _splat(arow, h0)
                    a1 = _splat(arow, h1)
                    msg_vs[s][e, pl.ds(0, 16)] = (
                        rows_vs[s][e, pl.ds(0, 16)] * a0)
                    msg_vs[s][e, pl.ds(16, 16)] = (
                        rows_vs[s][e, pl.ds(16, 16)] * a1)

            idx_issue(jnp.int32(0))
            idx_wait(jnp.int32(0))
            g_issue(jnp.int32(0), 0)
            g_issue(jnp.int32(1), 1)

            def pair_body(jj, carry):
                j0 = jj * 2
                j1 = j0 + 1

                @pl.when(jj >= 1)
                def _():
                    o_wait(j0 - 2, 0)
                    o_wait(j0 - 1, 1)

                @pl.when(jj < NCHP - 1)
                def _():
                    idx_issue(j0 + 2)
                g_wait(j0, 0)
                compute(0)
                o_issue(j0, 0)

                @pl.when(jj < NCHP - 1)
                def _():
                    idx_wait(j0 + 2)
                    g_issue(j0 + 2, 0)
                g_wait(j1, 1)
                compute(1)
                o_issue(j1, 1)

                @pl.when(jj < NCHP - 1)
                def _():
                    g_issue(j0 + 3, 1)
                return carry
            lax.fori_loop(0, NCHP, pair_body, 0)
            o_wait(jnp.int32(NCH - 2), 0)
            o_wait(jnp.int32(NCH - 1), 1)
            plsc.subcore_barrier()

            pltpu.sync_copy(b_h.at[c], bias_v)
            bv0 = bias_v[pl.ds(0, 16)]
            bv1 = bias_v[pl.ds(16, 16)]

            def wout_body(rc, carry):
                r0 = rb + rc * RC
                pltpu.sync_copy(acc.at[pl.ds(r0, RC)], msg_v0)

                def wrow(r, carry2):
                    v0 = msg_v0[r, pl.ds(0, 16)] + bv0
                    v1 = msg_v0[r, pl.ds(16, 16)] + bv1
                    if apply_relu:
                        v0 = jnp.maximum(v0, 0.0)
                        v1 = jnp.maximum(v1, 0.0)
                    msg_v0[r, pl.ds(0, 16)] = v0
                    msg_v0[r, pl.ds(16, 16)] = v1
                    return carry2
                lax.fori_loop(0, RC, wrow, 0)
                pltpu.sync_copy(msg_v0, out_h.at[pl.ds(r0, RC)])
                return carry
            lax.fori_loop(0, NRC, wout_body, 0)

        pl.when(cid == 0)(lambda: core_path(0))
        pl.when(cid == 1)(lambda: core_path(1))

    return _sc_b


_sc_b_relu = _make_sc_b(True)
_sc_b_plain = _make_sc_b(False)


def kernel(des, tweet, num_prop, cat_prop, edge_index,
           W_des, b_des, W_tw, b_tw, W_np, b_np, W_cp, b_cp,
           W_in, b_in, W1, a_src1, a_dst1, b1, W2, a_src2, a_dst2, b2):
    np_p = jnp.pad(num_prop, ((0, 0), (0, 3)))
    cp_p = jnp.pad(cat_prop, ((0, 0), (0, 7)))
    Wn_p = jnp.pad(W_np, ((0, 3), (0, 0)))
    Wc_p = jnp.pad(W_cp, ((0, 7), (0, 0)))

    eye4 = jnp.eye(4, dtype=F32)
    As1 = jnp.pad((a_src1[:, :, None] * eye4[:, None, :]).reshape(64, 4),
                  ((0, 0), (0, HW - 4)))
    Ad1 = jnp.pad((a_dst1[:, :, None] * eye4[:, None, :]).reshape(64, 4),
                  ((0, 0), (0, HW - 4)))
    As2 = jnp.pad(jnp.tile(a_src2.reshape(64, 1), (1, 4)),
                  ((0, 0), (0, HW - 4)))
    Ad2 = jnp.pad(jnp.tile(a_dst2.reshape(64, 1), (1, 4)),
                  ((0, 0), (0, HW - 4)))

    src = jnp.pad(edge_index[0], (0, EPAD - E),
                  constant_values=N).reshape(EPAD // CHUNK, CHUNK)
    dst = jnp.pad(edge_index[1], (0, EPAD - E),
                  constant_values=N).reshape(EPAD // CHUNK, CHUNK)

    b1_st = b1.reshape(2, 32)
    b2_st = b2.reshape(2, 32)

    xw1_st, als1, ald1, lee1 = _tc1(
        des, tweet, np_p, cp_p,
        W_des, b_des.reshape(1, 16), W_tw, b_tw.reshape(1, 16),
        Wn_p, b_np.reshape(1, 16), Wc_p, b_cp.reshape(1, 16),
        W_in, b_in.reshape(1, 64), W1, As1, Ad1)

    rden1, ee1 = _sc_a(src, dst, als1, ald1, lee1)
    x2_st = _sc_b_relu(src, dst, xw1_st, ee1, rden1, lee1, b1_st)

    xw2_st, als2, ald2, lee2 = _tc2(x2_st, W2, As2, Ad2)
    rden2, ee2 = _sc_a(src, dst, als2, ald2, lee2)
    z_st = _sc_b_plain(src, dst, xw2_st, ee2, rden2, lee2, b2_st)

    return jnp.concatenate([z_st[0, :N], z_st[1, :N]], axis=1)
